# Initial kernel scaffold; baseline (speedup 1.0000x reference)
import jax
import jax.numpy as jnp
from jax.experimental import pallas as pl


def kernel(users, posts, post_adjs, up_masking, W_user, b_user, W_post, b_post, W_gat0, a1_0, a2_0, W_gat1, a1_1, a2_1, Wp0, bp0, Wp1, bp1, Wu0, bu0, Wu1, bu1):
    raise NotImplementedError("write your pallas kernel here")


# speedup vs baseline: 1.4024x; 1.4024x over previous
import functools

import jax
import jax.numpy as jnp
from jax import lax
from jax.experimental import pallas as pl

N_USERS = 1024
N_POSTS = 5000
ROW_BLK = 256
USER_BLK = 256
_GRID_POSTS = (N_POSTS + ROW_BLK - 1) // ROW_BLK
_GRID_USERS = N_USERS // USER_BLK


def _leaky_relu(x):
    return jnp.where(x >= 0, x, 0.2 * x)


def _elu(x):
    return jnp.where(x > 0, x, jnp.exp(jnp.minimum(x, 0.0)) - 1.0)


def _encode_kernel(posts_ref, w_ref, b_ref, out_ref):
    out_ref[...] = jnp.dot(posts_ref[...], w_ref[...],
                           preferred_element_type=jnp.float32) + b_ref[...]


def _gat_rows(mask, hw_blk, hw_full, a1_ref, a2_ref):
    s1 = jnp.dot(hw_blk, a1_ref[...], preferred_element_type=jnp.float32)
    s2 = lax.dot_general(a2_ref[...], hw_full,
                         (((1,), (1,)), ((), ())),
                         preferred_element_type=jnp.float32)
    e = _leaky_relu(s1 + s2)
    e = jnp.where(mask, e, jnp.float32(-1e9))
    m = jnp.max(e, axis=1, keepdims=True)
    ex = jnp.exp(e - m)
    l = jnp.sum(ex, axis=1, keepdims=True)
    acc = jnp.dot(ex, hw_full, preferred_element_type=jnp.float32)
    return _elu(acc / l)


def _gat1_kernel(adj_ref, hw_blk_ref, hw_full_ref, a1_ref, a2_ref, wg1_ref,
                 hw1_ref, mask_ref):
    mask = adj_ref[...] > 0
    p1 = _gat_rows(mask, hw_blk_ref[...], hw_full_ref[...], a1_ref, a2_ref)
    hw1_ref[...] = jnp.dot(p1, wg1_ref[...], preferred_element_type=jnp.float32)
    mask_ref[...] = mask.astype(jnp.int8)


def _gat2_kernel(mask_ref, hw_blk_ref, hw_full_ref, a1_ref, a2_ref,
                 wp0_ref, bp0_ref, wp1_ref, bp1_ref,
                 p2_ref, label_ref):
    mask = mask_ref[...].astype(jnp.int32) > 0
    p2 = _gat_rows(mask, hw_blk_ref[...], hw_full_ref[...], a1_ref, a2_ref)
    p2_ref[...] = p2
    t = jnp.maximum(
        jnp.dot(p2, wp0_ref[...], preferred_element_type=jnp.float32)
        + bp0_ref[...], 0.0)
    label_ref[...] = (jnp.dot(t, wp1_ref[...],
                              preferred_element_type=jnp.float32)
                      + bp1_ref[...])


def _user_kernel(users_ref, up_ref, p2_ref, wu_ref, bu_ref,
                 wu0a_ref, wu0b_ref, bu0_ref, wu1_ref, bu1_ref, out_ref):
    up = up_ref[...]
    u = jnp.dot(users_ref[...], wu_ref[...],
                preferred_element_type=jnp.float32) + bu_ref[...]
    denom = jnp.sum(up, axis=1, keepdims=True) + 1e-9
    agg = jnp.dot(up, p2_ref[...], preferred_element_type=jnp.float32) / denom
    h = jnp.maximum(
        jnp.dot(u, wu0a_ref[...], preferred_element_type=jnp.float32)
        + jnp.dot(agg, wu0b_ref[...], preferred_element_type=jnp.float32)
        + bu0_ref[...], 0.0)
    out_ref[...] = (jnp.dot(h, wu1_ref[...],
                            preferred_element_type=jnp.float32)
                    + bu1_ref[...])


def _full(shape):
    return pl.BlockSpec(shape, lambda i: (0,) * len(shape))


def _rows(ncols, blk=ROW_BLK):
    return pl.BlockSpec((blk, ncols), lambda i: (i, 0))


@jax.jit
def kernel(users, posts, post_adjs, up_masking, W_user, b_user, W_post, b_post,
           W_gat0, a1_0, a2_0, W_gat1, a1_1, a2_1,
           Wp0, bp0, Wp1, bp1, Wu0, bu0, Wu1, bu1):
    f32 = jnp.float32
    D = W_gat0.shape[0]

    w_enc = W_post @ W_gat0
    b_enc = (b_post @ W_gat0).reshape(1, D)

    hw0 = pl.pallas_call(
        _encode_kernel,
        grid=(_GRID_POSTS,),
        in_specs=[_rows(posts.shape[1]), _full(w_enc.shape), _full((1, D))],
        out_specs=_rows(D),
        out_shape=jax.ShapeDtypeStruct((N_POSTS, D), f32),
    )(posts, w_enc, b_enc)

    hw1, mask8 = pl.pallas_call(
        _gat1_kernel,
        grid=(_GRID_POSTS,),
        in_specs=[_rows(N_POSTS), _rows(D), _full((N_POSTS, D)),
                  _full((D, 1)), _full((1, D)), _full((D, D))],
        out_specs=[_rows(D), _rows(N_POSTS)],
        out_shape=[jax.ShapeDtypeStruct((N_POSTS, D), f32),
                   jax.ShapeDtypeStruct((N_POSTS, N_POSTS), jnp.int8)],
    )(post_adjs, hw0, hw0, a1_0.reshape(D, 1), a2_0.reshape(1, D), W_gat1)

    p2, post_label = pl.pallas_call(
        _gat2_kernel,
        grid=(_GRID_POSTS,),
        in_specs=[_rows(N_POSTS), _rows(D), _full((N_POSTS, D)),
                  _full((D, 1)), _full((1, D)),
                  _full(Wp0.shape), _full((1, Wp0.shape[1])),
                  _full(Wp1.shape), _full((1, 1))],
        out_specs=[_rows(D), _rows(1)],
        out_shape=[jax.ShapeDtypeStruct((N_POSTS, D), f32),
                   jax.ShapeDtypeStruct((N_POSTS, 1), f32)],
    )(mask8, hw1, hw1, a1_1.reshape(D, 1), a2_1.reshape(1, D),
      Wp0, bp0.reshape(1, -1), Wp1, bp1.reshape(1, 1))

    d_ue = W_user.shape[1]
    user_label = pl.pallas_call(
        _user_kernel,
        grid=(_GRID_USERS,),
        in_specs=[_rows(users.shape[1], USER_BLK), _rows(N_POSTS, USER_BLK),
                  _full((N_POSTS, D)),
                  _full(W_user.shape), _full((1, d_ue)),
                  _full((d_ue, Wu0.shape[1])), _full((D, Wu0.shape[1])),
                  _full((1, Wu0.shape[1])), _full(Wu1.shape), _full((1, 1))],
        out_specs=_rows(1, USER_BLK),
        out_shape=jax.ShapeDtypeStruct((N_USERS, 1), f32),
    )(users, up_masking, p2, W_user, b_user.reshape(1, -1),
      Wu0[:d_ue], Wu0[d_ue:], bu0.reshape(1, -1), Wu1, bu1.reshape(1, 1))

    return (user_label, post_label)
